# baseline (speedup 1.0000x reference)
import functools
import math

import jax
import jax.numpy as jnp
from jax import lax
from jax.experimental import pallas as pl
from jax.experimental.pallas import tpu as pltpu

N = 10000
E = 320000
D_IN = 128
MID = 16
OUT = 16
NEG = -jnp.inf


def _elu(v):
    return jnp.where(v > 0, v, jnp.exp(jnp.minimum(v, 0.0)) - 1.0)


def _k1_body(x_ref, w_ref, b_ref, o_ref):
    o_ref[...] = (
        jnp.dot(x_ref[...], w_ref[...], preferred_element_type=jnp.float32)
        + b_ref[...]
    )


def _node_proj(x, Wn, bn):
    B = 2000
    return pl.pallas_call(
        _k1_body,
        grid=(N // B,),
        in_specs=[
            pl.BlockSpec((B, D_IN), lambda i: (i, 0)),
            pl.BlockSpec((D_IN, MID), lambda i: (0, 0)),
            pl.BlockSpec((1, MID), lambda i: (0, 0)),
        ],
        out_specs=pl.BlockSpec((B, MID), lambda i: (i, 0)),
        out_shape=jax.ShapeDtypeStruct((N, MID), jnp.float32),
    )(x, Wn, bn.reshape(1, MID))


def _k2_body(g_ref, a_ref, we_ref, be_ref, wm_ref, bm_ref, o_ref):
    t = (
        g_ref[...]
        + jnp.dot(a_ref[...], we_ref[...], preferred_element_type=jnp.float32)
        + be_ref[...]
    )
    t = jnp.maximum(t, 0.0)
    o_ref[...] = (
        jnp.dot(t, wm_ref[...], preferred_element_type=jnp.float32) + bm_ref[...]
    )


def _edge_mlp(g, attr, We, be, Wm, bm):
    g2 = g.reshape(E // 8, 128)
    a2 = attr.reshape(E // 8, 32)
    eye = jnp.eye(8, dtype=jnp.float32)
    WeB = jnp.einsum("pq,ij->piqj", eye, We).reshape(32, 128)
    WmB = jnp.einsum("pq,ij->piqj", eye, Wm).reshape(128, 128)
    beB = jnp.tile(be, 8).reshape(1, 128)
    bmB = jnp.tile(bm, 8).reshape(1, 128)
    B = 4000
    out = pl.pallas_call(
        _k2_body,
        grid=(E // 8 // B,),
        in_specs=[
            pl.BlockSpec((B, 128), lambda i: (i, 0)),
            pl.BlockSpec((B, 32), lambda i: (i, 0)),
            pl.BlockSpec((32, 128), lambda i: (0, 0)),
            pl.BlockSpec((1, 128), lambda i: (0, 0)),
            pl.BlockSpec((128, 128), lambda i: (0, 0)),
            pl.BlockSpec((1, 128), lambda i: (0, 0)),
        ],
        out_specs=pl.BlockSpec((B, 128), lambda i: (i, 0)),
        out_shape=jax.ShapeDtypeStruct((E // 8, 128), jnp.float32),
    )(g2, a2, WeB, beB, WmB, bmB)
    return out.reshape(E, MID)


def _k3_body(parts_ref, wo_ref, bo_ref, w2_ref, b2_ref, o_ref, *, mode):
    agg = jnp.max(parts_ref[...], axis=0)
    agg = jnp.where(jnp.isfinite(agg), agg, 0.0)
    out = (
        jnp.dot(agg, wo_ref[...], preferred_element_type=jnp.float32) + bo_ref[...]
    )
    if mode == 0:
        h = _elu(_elu(out))
        o_ref[...] = (
            jnp.dot(h, w2_ref[...], preferred_element_type=jnp.float32)
            + b2_ref[...]
        )
    else:
        m = jnp.max(out, axis=1, keepdims=True)
        s = out - m
        lse = jnp.log(jnp.sum(jnp.exp(s), axis=1, keepdims=True))
        o_ref[...] = s - lse


def _post_agg(parts, Wo, bo, W2, b2, mode):
    P = parts.shape[0]
    B = 2000
    dout = W2.shape[1] if mode == 0 else OUT
    return pl.pallas_call(
        functools.partial(_k3_body, mode=mode),
        grid=(N // B,),
        in_specs=[
            pl.BlockSpec((P, B, MID), lambda i: (0, i, 0)),
            pl.BlockSpec((MID, MID), lambda i: (0, 0)),
            pl.BlockSpec((1, MID), lambda i: (0, 0)),
            pl.BlockSpec((MID, dout), lambda i: (0, 0)),
            pl.BlockSpec((1, dout), lambda i: (0, 0)),
        ],
        out_specs=pl.BlockSpec((B, dout), lambda i: (i, 0)),
        out_shape=jax.ShapeDtypeStruct((N, dout), jnp.float32),
    )(parts, Wo, bo.reshape(1, MID), W2, b2.reshape(1, dout))


def _gather_rows(table, src):
    return table[src]


def _scatter_max(msg, dst):
    agg = jax.ops.segment_max(msg, dst, num_segments=N)
    return agg.reshape(1, N, MID)


def kernel(x, edge_index, edge_attr, W1_node, b1_node, W1_edge, b1_edge,
           W1_net, b1_net, W1_out, b1_out, W2_node, b2_node, W2_edge, b2_edge,
           W2_net, b2_net, W2_out, b2_out):
    src, dst = edge_index[0], edge_index[1]

    h1 = _node_proj(x, W1_node, b1_node)
    g1 = _gather_rows(h1, src)
    msg1 = _edge_mlp(g1, edge_attr, W1_edge, b1_edge, W1_net, b1_net)
    parts1 = _scatter_max(msg1, dst)
    h2 = _post_agg(parts1, W1_out, b1_out, W2_node, b2_node, mode=0)

    g2 = _gather_rows(h2, src)
    msg2 = _edge_mlp(g2, edge_attr, W2_edge, b2_edge, W2_net, b2_net)
    parts2 = _scatter_max(msg2, dst)
    out = _post_agg(parts2, W2_out, b2_out, W2_out, b2_out, mode=1)
    return out


# speedup vs baseline: 1.6070x; 1.6070x over previous
import functools
import math

import jax
import jax.numpy as jnp
from jax import lax
from jax.experimental import pallas as pl
from jax.experimental.pallas import tpu as pltpu
from jax.experimental.pallas import tpu_sc as plsc

N = 10000
E = 320000
D_IN = 128
MID = 16
OUT = 16
NEG = -jnp.inf


def _elu(v):
    return jnp.where(v > 0, v, jnp.exp(jnp.minimum(v, 0.0)) - 1.0)


def _k1_body(x_ref, w_ref, b_ref, o_ref):
    o_ref[...] = (
        jnp.dot(x_ref[...], w_ref[...], preferred_element_type=jnp.float32)
        + b_ref[...]
    )


def _node_proj(x, Wn, bn):
    B = 2000
    return pl.pallas_call(
        _k1_body,
        grid=(N // B,),
        in_specs=[
            pl.BlockSpec((B, D_IN), lambda i: (i, 0)),
            pl.BlockSpec((D_IN, MID), lambda i: (0, 0)),
            pl.BlockSpec((1, MID), lambda i: (0, 0)),
        ],
        out_specs=pl.BlockSpec((B, MID), lambda i: (i, 0)),
        out_shape=jax.ShapeDtypeStruct((N, MID), jnp.float32),
    )(x, Wn, bn.reshape(1, MID))


def _k2_body(g_ref, a_ref, we_ref, be_ref, wm_ref, bm_ref, o_ref):
    t = (
        g_ref[...]
        + jnp.dot(a_ref[...], we_ref[...], preferred_element_type=jnp.float32)
        + be_ref[...]
    )
    t = jnp.maximum(t, 0.0)
    o_ref[...] = (
        jnp.dot(t, wm_ref[...], preferred_element_type=jnp.float32) + bm_ref[...]
    )


def _edge_mlp(g, attr, We, be, Wm, bm):
    g2 = g.reshape(E // 8, 128)
    a2 = attr.reshape(E // 8, 32)
    eye = jnp.eye(8, dtype=jnp.float32)
    WeB = jnp.einsum("pq,ij->piqj", eye, We).reshape(32, 128)
    WmB = jnp.einsum("pq,ij->piqj", eye, Wm).reshape(128, 128)
    beB = jnp.tile(be, 8).reshape(1, 128)
    bmB = jnp.tile(bm, 8).reshape(1, 128)
    B = 4000
    out = pl.pallas_call(
        _k2_body,
        grid=(E // 8 // B,),
        in_specs=[
            pl.BlockSpec((B, 128), lambda i: (i, 0)),
            pl.BlockSpec((B, 32), lambda i: (i, 0)),
            pl.BlockSpec((32, 128), lambda i: (0, 0)),
            pl.BlockSpec((1, 128), lambda i: (0, 0)),
            pl.BlockSpec((128, 128), lambda i: (0, 0)),
            pl.BlockSpec((1, 128), lambda i: (0, 0)),
        ],
        out_specs=pl.BlockSpec((B, 128), lambda i: (i, 0)),
        out_shape=jax.ShapeDtypeStruct((E // 8, 128), jnp.float32),
    )(g2, a2, WeB, beB, WmB, bmB)
    return out.reshape(E, MID)


def _k3_body(parts_ref, wo_ref, bo_ref, w2_ref, b2_ref, o_ref, *, mode):
    agg = jnp.max(parts_ref[...], axis=0)
    agg = jnp.where(jnp.isfinite(agg), agg, 0.0)
    out = (
        jnp.dot(agg, wo_ref[...], preferred_element_type=jnp.float32) + bo_ref[...]
    )
    if mode == 0:
        h = _elu(_elu(out))
        o_ref[...] = (
            jnp.dot(h, w2_ref[...], preferred_element_type=jnp.float32)
            + b2_ref[...]
        )
    else:
        m = jnp.max(out, axis=1, keepdims=True)
        s = out - m
        lse = jnp.log(jnp.sum(jnp.exp(s), axis=1, keepdims=True))
        o_ref[...] = s - lse


def _post_agg(parts, Wo, bo, W2, b2, mode):
    P = parts.shape[0]
    B = 2000
    dout = W2.shape[1] if mode == 0 else OUT
    return pl.pallas_call(
        functools.partial(_k3_body, mode=mode),
        grid=(N // B,),
        in_specs=[
            pl.BlockSpec((P, B, MID), lambda i: (0, i, 0)),
            pl.BlockSpec((MID, MID), lambda i: (0, 0)),
            pl.BlockSpec((1, MID), lambda i: (0, 0)),
            pl.BlockSpec((MID, dout), lambda i: (0, 0)),
            pl.BlockSpec((1, dout), lambda i: (0, 0)),
        ],
        out_specs=pl.BlockSpec((B, dout), lambda i: (i, 0)),
        out_shape=jax.ShapeDtypeStruct((N, dout), jnp.float32),
    )(parts, Wo, bo.reshape(1, MID), W2, b2.reshape(1, dout))


_SC_MESH = plsc.VectorSubcoreMesh(core_axis_name="c", subcore_axis_name="s")
_NW = 32
_EPW = E // _NW
_GC = 2000


def _gather_body(table_hbm, src_hbm, out_hbm, idx_v, rows_v, sem):
    wid = lax.axis_index("s") * 2 + lax.axis_index("c")
    base = wid * _EPW

    def chunk(i, _):
        off = base + i * _GC
        pltpu.sync_copy(src_hbm.at[pl.ds(off, _GC)], idx_v)
        pltpu.async_copy(table_hbm.at[idx_v], rows_v, sem).wait()
        pltpu.sync_copy(rows_v, out_hbm.at[pl.ds(off, _GC)])
        return 0

    lax.fori_loop(0, _EPW // _GC, chunk, 0)


def _gather_rows(table, src):
    f = pl.kernel(
        _gather_body,
        out_type=jax.ShapeDtypeStruct((E, MID), jnp.float32),
        mesh=_SC_MESH,
        scratch_types=[
            pltpu.VMEM((_GC,), jnp.int32),
            pltpu.VMEM((_GC, MID), jnp.float32),
            pltpu.SemaphoreType.DMA,
        ],
        compiler_params=pltpu.CompilerParams(use_tc_tiling_on_sc=False),
    )
    return f(table, src)


def _scatter_max(msg, dst):
    agg = jax.ops.segment_max(msg, dst, num_segments=N)
    return agg.reshape(1, N, MID)


def kernel(x, edge_index, edge_attr, W1_node, b1_node, W1_edge, b1_edge,
           W1_net, b1_net, W1_out, b1_out, W2_node, b2_node, W2_edge, b2_edge,
           W2_net, b2_net, W2_out, b2_out):
    src, dst = edge_index[0], edge_index[1]

    h1 = _node_proj(x, W1_node, b1_node)
    g1 = _gather_rows(h1, src)
    msg1 = _edge_mlp(g1, edge_attr, W1_edge, b1_edge, W1_net, b1_net)
    parts1 = _scatter_max(msg1, dst)
    h2 = _post_agg(parts1, W1_out, b1_out, W2_node, b2_node, mode=0)

    g2 = _gather_rows(h2, src)
    msg2 = _edge_mlp(g2, edge_attr, W2_edge, b2_edge, W2_net, b2_net)
    parts2 = _scatter_max(msg2, dst)
    out = _post_agg(parts2, W2_out, b2_out, W2_out, b2_out, mode=1)
    return out


# speedup vs baseline: 3.8622x; 2.4034x over previous
import functools
import math

import jax
import jax.numpy as jnp
from jax import lax
from jax.experimental import pallas as pl
from jax.experimental.pallas import tpu as pltpu
from jax.experimental.pallas import tpu_sc as plsc

N = 10000
E = 320000
D_IN = 128
MID = 16
OUT = 16
NEG = -jnp.inf


def _elu(v):
    return jnp.where(v > 0, v, jnp.exp(jnp.minimum(v, 0.0)) - 1.0)


def _k1_body(x_ref, w_ref, b_ref, o_ref):
    o_ref[...] = (
        jnp.dot(x_ref[...], w_ref[...], preferred_element_type=jnp.float32)
        + b_ref[...]
    )


def _node_proj(x, Wn, bn):
    B = 2000
    return pl.pallas_call(
        _k1_body,
        grid=(N // B,),
        in_specs=[
            pl.BlockSpec((B, D_IN), lambda i: (i, 0)),
            pl.BlockSpec((D_IN, MID), lambda i: (0, 0)),
            pl.BlockSpec((1, MID), lambda i: (0, 0)),
        ],
        out_specs=pl.BlockSpec((B, MID), lambda i: (i, 0)),
        out_shape=jax.ShapeDtypeStruct((N, MID), jnp.float32),
    )(x, Wn, bn.reshape(1, MID))


def _k2_body(g_ref, a_ref, we_ref, be_ref, wm_ref, bm_ref, o_ref):
    t = (
        g_ref[...]
        + jnp.dot(a_ref[...], we_ref[...], preferred_element_type=jnp.float32)
        + be_ref[...]
    )
    t = jnp.maximum(t, 0.0)
    o_ref[...] = (
        jnp.dot(t, wm_ref[...], preferred_element_type=jnp.float32) + bm_ref[...]
    )


def _edge_mlp(g, attr, We, be, Wm, bm):
    g2 = g.reshape(E // 8, 128)
    a2 = attr.reshape(E // 8, 32)
    eye = jnp.eye(8, dtype=jnp.float32)
    WeB = jnp.einsum("pq,ij->piqj", eye, We).reshape(32, 128)
    WmB = jnp.einsum("pq,ij->piqj", eye, Wm).reshape(128, 128)
    beB = jnp.tile(be, 8).reshape(1, 128)
    bmB = jnp.tile(bm, 8).reshape(1, 128)
    B = 4000
    out = pl.pallas_call(
        _k2_body,
        grid=(E // 8 // B,),
        in_specs=[
            pl.BlockSpec((B, 128), lambda i: (i, 0)),
            pl.BlockSpec((B, 32), lambda i: (i, 0)),
            pl.BlockSpec((32, 128), lambda i: (0, 0)),
            pl.BlockSpec((1, 128), lambda i: (0, 0)),
            pl.BlockSpec((128, 128), lambda i: (0, 0)),
            pl.BlockSpec((1, 128), lambda i: (0, 0)),
        ],
        out_specs=pl.BlockSpec((B, 128), lambda i: (i, 0)),
        out_shape=jax.ShapeDtypeStruct((E // 8, 128), jnp.float32),
    )(g2, a2, WeB, beB, WmB, bmB)
    return out.reshape(E, MID)


def _k3_body(parts_ref, wo_ref, bo_ref, w2_ref, b2_ref, o_ref, *, mode):
    agg = jnp.max(parts_ref[...], axis=0)
    agg = jnp.where(jnp.isfinite(agg), agg, 0.0)
    out = (
        jnp.dot(agg, wo_ref[...], preferred_element_type=jnp.float32) + bo_ref[...]
    )
    if mode == 0:
        h = _elu(_elu(out))
        o_ref[...] = (
            jnp.dot(h, w2_ref[...], preferred_element_type=jnp.float32)
            + b2_ref[...]
        )
    else:
        m = jnp.max(out, axis=1, keepdims=True)
        s = out - m
        lse = jnp.log(jnp.sum(jnp.exp(s), axis=1, keepdims=True))
        o_ref[...] = s - lse


def _post_agg(parts, Wo, bo, W2, b2, mode):
    P = parts.shape[0]
    B = 2000
    dout = W2.shape[1] if mode == 0 else OUT
    return pl.pallas_call(
        functools.partial(_k3_body, mode=mode),
        grid=(N // B,),
        in_specs=[
            pl.BlockSpec((P, B, MID), lambda i: (0, i, 0)),
            pl.BlockSpec((MID, MID), lambda i: (0, 0)),
            pl.BlockSpec((1, MID), lambda i: (0, 0)),
            pl.BlockSpec((MID, dout), lambda i: (0, 0)),
            pl.BlockSpec((1, dout), lambda i: (0, 0)),
        ],
        out_specs=pl.BlockSpec((B, dout), lambda i: (i, 0)),
        out_shape=jax.ShapeDtypeStruct((N, dout), jnp.float32),
    )(parts, Wo, bo.reshape(1, MID), W2, b2.reshape(1, dout))


_SC_MESH = plsc.VectorSubcoreMesh(core_axis_name="c", subcore_axis_name="s")
_NW = 32
_EPW = E // _NW
_GC = 2000


def _gather_body(table_hbm, src_hbm, out_hbm, idx_v, rows_v, sem):
    wid = lax.axis_index("s") * 2 + lax.axis_index("c")
    base = wid * _EPW

    def chunk(i, _):
        off = base + i * _GC
        pltpu.sync_copy(src_hbm.at[pl.ds(off, _GC)], idx_v)
        pltpu.async_copy(table_hbm.at[idx_v], rows_v, sem).wait()
        pltpu.sync_copy(rows_v, out_hbm.at[pl.ds(off, _GC)])
        return 0

    lax.fori_loop(0, _EPW // _GC, chunk, 0)


def _gather_rows(table, src):
    f = pl.kernel(
        _gather_body,
        out_type=jax.ShapeDtypeStruct((E, MID), jnp.float32),
        mesh=_SC_MESH,
        scratch_types=[
            pltpu.VMEM((_GC,), jnp.int32),
            pltpu.VMEM((_GC, MID), jnp.float32),
            pltpu.SemaphoreType.DMA,
        ],
        compiler_params=pltpu.CompilerParams(use_tc_tiling_on_sc=False),
    )
    return f(table, src)


_HALF = 5120
_NPAD = 2 * _HALF
_STRIPE = 640
_RPT = _STRIPE // 16
_SCC = 1000


def _scatter_body(msg_hbm, dst_hbm, out_hbm, part_v, dstc_v, msgc_v,
                  acc_v, red_v, shared):
    c = lax.axis_index("c")
    sid = lax.axis_index("s")
    wid = sid * 2 + c
    base = wid * _EPW
    vneg = jnp.full((MID,), NEG, jnp.float32)

    for p in range(2):
        def initrow(i, _):
            part_v[i] = vneg
            return 0
        lax.fori_loop(0, _HALF + 16, initrow, 0)

        def chunk(i, _):
            off = base + i * _SCC
            pltpu.sync_copy(dst_hbm.at[pl.ds(off, _SCC)], dstc_v)
            pltpu.sync_copy(msg_hbm.at[pl.ds(off, _SCC)], msgc_v)

            def group(g, _):
                dvec = dstc_v[pl.ds(g * 16, 16)] - p * _HALF
                okv = jnp.logical_and(dvec >= 0, dvec < _HALF)
                slotv = jnp.where(okv, dvec, _HALF)
                for l in range(16):
                    slot = slotv[l]
                    part_v[slot] = jnp.maximum(
                        part_v[slot], msgc_v[g * 16 + l])
                return 0

            lax.fori_loop(0, _SCC // 16, group, 0)
            return 0

        lax.fori_loop(0, _EPW // _SCC, chunk, 0)

        for r in range(_HALF // _STRIPE):
            pltpu.sync_copy(
                part_v.at[pl.ds(r * _STRIPE, _STRIPE)], shared.at[sid])
            plsc.subcore_barrier()
            pltpu.sync_copy(shared.at[0, pl.ds(sid * _RPT, _RPT)], acc_v)
            for t in range(1, 16):
                pltpu.sync_copy(shared.at[t, pl.ds(sid * _RPT, _RPT)], red_v)

                def redrow(q, _):
                    acc_v[q] = jnp.maximum(acc_v[q], red_v[q])
                    return 0
                lax.fori_loop(0, _RPT, redrow, 0)
            pltpu.sync_copy(
                acc_v,
                out_hbm.at[c, pl.ds(p * _HALF + r * _STRIPE + sid * _RPT, _RPT)])
            plsc.subcore_barrier()


def _scatter_max(msg, dst):
    f = pl.kernel(
        _scatter_body,
        out_type=jax.ShapeDtypeStruct((2, _NPAD, MID), jnp.float32),
        mesh=_SC_MESH,
        scratch_types=[
            pltpu.VMEM((_HALF + 16, MID), jnp.float32),
            pltpu.VMEM((_SCC,), jnp.int32),
            pltpu.VMEM((_SCC, MID), jnp.float32),
            pltpu.VMEM((_RPT, MID), jnp.float32),
            pltpu.VMEM((_RPT, MID), jnp.float32),
            pltpu.VMEM_SHARED((16, _STRIPE, MID), jnp.float32),
        ],
        compiler_params=pltpu.CompilerParams(use_tc_tiling_on_sc=False),
    )
    return f(msg, dst)


def kernel(x, edge_index, edge_attr, W1_node, b1_node, W1_edge, b1_edge,
           W1_net, b1_net, W1_out, b1_out, W2_node, b2_node, W2_edge, b2_edge,
           W2_net, b2_net, W2_out, b2_out):
    src, dst = edge_index[0], edge_index[1]

    h1 = _node_proj(x, W1_node, b1_node)
    g1 = _gather_rows(h1, src)
    msg1 = _edge_mlp(g1, edge_attr, W1_edge, b1_edge, W1_net, b1_net)
    parts1 = _scatter_max(msg1, dst)
    h2 = _post_agg(parts1, W1_out, b1_out, W2_node, b2_node, mode=0)

    g2 = _gather_rows(h2, src)
    msg2 = _edge_mlp(g2, edge_attr, W2_edge, b2_edge, W2_net, b2_net)
    parts2 = _scatter_max(msg2, dst)
    out = _post_agg(parts2, W2_out, b2_out, W2_out, b2_out, mode=1)
    return out


# speedup vs baseline: 4.5394x; 1.1753x over previous
import functools
import math

import jax
import jax.numpy as jnp
from jax import lax
from jax.experimental import pallas as pl
from jax.experimental.pallas import tpu as pltpu
from jax.experimental.pallas import tpu_sc as plsc

N = 10000
E = 320000
D_IN = 128
MID = 16
OUT = 16
NEG = -jnp.inf


def _elu(v):
    return jnp.where(v > 0, v, jnp.exp(jnp.minimum(v, 0.0)) - 1.0)


def _k1_body(x_ref, w_ref, b_ref, o_ref):
    o_ref[...] = (
        jnp.dot(x_ref[...], w_ref[...], preferred_element_type=jnp.float32)
        + b_ref[...]
    )


def _node_proj(x, Wn, bn):
    B = 2000
    return pl.pallas_call(
        _k1_body,
        grid=(N // B,),
        in_specs=[
            pl.BlockSpec((B, D_IN), lambda i: (i, 0)),
            pl.BlockSpec((D_IN, MID), lambda i: (0, 0)),
            pl.BlockSpec((1, MID), lambda i: (0, 0)),
        ],
        out_specs=pl.BlockSpec((B, MID), lambda i: (i, 0)),
        out_shape=jax.ShapeDtypeStruct((N, MID), jnp.float32),
    )(x, Wn, bn.reshape(1, MID))


def _k2_body(g_ref, a_ref, we_ref, be_ref, wm_ref, bm_ref, o_ref):
    t = (
        g_ref[...]
        + jnp.dot(a_ref[...], we_ref[...], preferred_element_type=jnp.float32)
        + be_ref[...]
    )
    t = jnp.maximum(t, 0.0)
    o_ref[...] = (
        jnp.dot(t, wm_ref[...], preferred_element_type=jnp.float32) + bm_ref[...]
    )


def _edge_mlp(g, attr, We, be, Wm, bm):
    g2 = g.reshape(E // 8, 128)
    a2 = attr.reshape(E // 8, 32)
    eye = jnp.eye(8, dtype=jnp.float32)
    WeB = jnp.einsum("pq,ij->piqj", eye, We).reshape(32, 128)
    WmB = jnp.einsum("pq,ij->piqj", eye, Wm).reshape(128, 128)
    beB = jnp.tile(be, 8).reshape(1, 128)
    bmB = jnp.tile(bm, 8).reshape(1, 128)
    B = 4000
    out = pl.pallas_call(
        _k2_body,
        grid=(E // 8 // B,),
        in_specs=[
            pl.BlockSpec((B, 128), lambda i: (i, 0)),
            pl.BlockSpec((B, 32), lambda i: (i, 0)),
            pl.BlockSpec((32, 128), lambda i: (0, 0)),
            pl.BlockSpec((1, 128), lambda i: (0, 0)),
            pl.BlockSpec((128, 128), lambda i: (0, 0)),
            pl.BlockSpec((1, 128), lambda i: (0, 0)),
        ],
        out_specs=pl.BlockSpec((B, 128), lambda i: (i, 0)),
        out_shape=jax.ShapeDtypeStruct((E // 8, 128), jnp.float32),
    )(g2, a2, WeB, beB, WmB, bmB)
    return out.reshape(E, MID)


def _k3_body(parts_ref, wo_ref, bo_ref, w2_ref, b2_ref, o_ref, *, mode):
    p = parts_ref[...]
    agg = jnp.concatenate([p[0], p[1]], axis=1)
    agg = jnp.where(jnp.isfinite(agg), agg, 0.0)
    out = (
        jnp.dot(agg, wo_ref[...], preferred_element_type=jnp.float32) + bo_ref[...]
    )
    if mode == 0:
        h = _elu(_elu(out))
        o_ref[...] = (
            jnp.dot(h, w2_ref[...], preferred_element_type=jnp.float32)
            + b2_ref[...]
        )
    else:
        m = jnp.max(out, axis=1, keepdims=True)
        s = out - m
        lse = jnp.log(jnp.sum(jnp.exp(s), axis=1, keepdims=True))
        o_ref[...] = s - lse


def _post_agg(parts, Wo, bo, W2, b2, mode):
    B = 2000
    dout = W2.shape[1] if mode == 0 else OUT
    return pl.pallas_call(
        functools.partial(_k3_body, mode=mode),
        grid=(N // B,),
        in_specs=[
            pl.BlockSpec((2, B, MID // 2), lambda i: (0, i, 0)),
            pl.BlockSpec((MID, MID), lambda i: (0, 0)),
            pl.BlockSpec((1, MID), lambda i: (0, 0)),
            pl.BlockSpec((MID, dout), lambda i: (0, 0)),
            pl.BlockSpec((1, dout), lambda i: (0, 0)),
        ],
        out_specs=pl.BlockSpec((B, dout), lambda i: (i, 0)),
        out_shape=jax.ShapeDtypeStruct((N, dout), jnp.float32),
    )(parts, Wo, bo.reshape(1, MID), W2, b2.reshape(1, dout))


_SC_MESH = plsc.VectorSubcoreMesh(core_axis_name="c", subcore_axis_name="s")
_NW = 32
_EPW = E // _NW
_GC = 2000


def _gather_body(table_hbm, src_hbm, out_hbm, idx_v, rows_v, sem):
    wid = lax.axis_index("s") * 2 + lax.axis_index("c")
    base = wid * _EPW

    def chunk(i, _):
        off = base + i * _GC
        pltpu.sync_copy(src_hbm.at[pl.ds(off, _GC)], idx_v)
        pltpu.async_copy(table_hbm.at[idx_v], rows_v, sem).wait()
        pltpu.sync_copy(rows_v, out_hbm.at[pl.ds(off, _GC)])
        return 0

    lax.fori_loop(0, _EPW // _GC, chunk, 0)


def _gather_rows(table, src):
    f = pl.kernel(
        _gather_body,
        out_type=jax.ShapeDtypeStruct((E, MID), jnp.float32),
        mesh=_SC_MESH,
        scratch_types=[
            pltpu.VMEM((_GC,), jnp.int32),
            pltpu.VMEM((_GC, MID), jnp.float32),
            pltpu.SemaphoreType.DMA,
        ],
        compiler_params=pltpu.CompilerParams(use_tc_tiling_on_sc=False),
    )
    return f(table, src)


_NPAD = 10240
_STRIPE = 1024
_NPS = _STRIPE // 16
_SCC = 800
_EPT = E // 16
_FPC = MID // 2


def _scatter_body(msg_hbm, dst_hbm, out_hbm, part_v, dstc_v, msgc_v,
                  accf_v, redf_v, accfl_v, rm_v, shared):
    c = lax.axis_index("c")
    sid = lax.axis_index("s")
    base = sid * _EPT
    iot = lax.iota(jnp.int32, 16)
    vneg = jnp.full((16,), NEG, jnp.float32)

    def initcol(k, _):
        part_v[pl.ds(k * 16, 16)] = vneg
        return 0
    lax.fori_loop(0, _FPC * _NPAD // 16, initcol, 0)

    def chunk(i, _):
        off = base + i * _SCC
        pltpu.sync_copy(dst_hbm.at[pl.ds(off, _SCC)], dstc_v)
        pltpu.sync_copy(msg_hbm.at[pl.ds(off * MID, _SCC * MID)], msgc_v)

        def group(g, _):
            dvec = dstc_v[pl.ds(g * 16, 16)]
            eb = (iot + g * 16) * MID + c * _FPC
            msgf = [plsc.load_gather(msgc_v, [eb + f]) for f in range(_FPC)]
            srt, _ = plsc.sort_key_val(dvec, iot)
            adj = srt.at[jnp.minimum(iot + 1, 15)].get(
                mode="promise_in_bounds")
            eqv = jnp.logical_and(srt == adj, iot < 15)
            cnt = plsc.all_reduce_population_count(eqv)

            def fast():
                for f in range(_FPC):
                    pidx = dvec + f * _NPAD
                    cur = plsc.load_gather(part_v, [pidx])
                    plsc.store_scatter(part_v, [pidx],
                                       jnp.maximum(cur, msgf[f]))

            def slow():
                for l in range(16):
                    m = iot == l
                    for f in range(_FPC):
                        pidx = dvec + f * _NPAD
                        cur = plsc.load_gather(part_v, [pidx])
                        plsc.store_scatter(part_v, [pidx],
                                           jnp.maximum(cur, msgf[f]), mask=m)

            lax.cond(cnt[0] > 0, slow, fast)
            return 0

        lax.fori_loop(0, _SCC // 16, group, 0)
        return 0

    lax.fori_loop(0, _EPT // _SCC, chunk, 0)

    idx_t = (iot % _FPC) * _NPS + iot // _FPC

    def stripe(r, _):
        for f in range(_FPC):
            pltpu.sync_copy(
                part_v.at[pl.ds(f * _NPAD + r * _STRIPE, _STRIPE)],
                shared.at[sid, f])
        plsc.subcore_barrier()
        n0 = sid * _NPS
        pltpu.sync_copy(shared.at[0, :, pl.ds(n0, _NPS)], accf_v)

        def tred(t, _):
            pltpu.sync_copy(shared.at[t, :, pl.ds(n0, _NPS)], redf_v)
            for f in range(_FPC):
                for q in range(_NPS // 16):
                    sl = pl.ds(q * 16, 16)
                    accf_v[f, sl] = jnp.maximum(accf_v[f, sl], redf_v[f, sl])
            return 0
        lax.fori_loop(1, 16, tred, 0)

        for f in range(_FPC):
            for q in range(_NPS // 16):
                accfl_v[pl.ds(f * _NPS + q * 16, 16)] = \
                    accf_v[f, pl.ds(q * 16, 16)]
        for j in range(_NPS // 2):
            rm_v[pl.ds(j * 16, 16)] = plsc.load_gather(
                accfl_v, [idx_t + 2 * j])
        pltpu.sync_copy(
            rm_v,
            out_hbm.at[c, pl.ds((r * _STRIPE + n0) * _FPC, _NPS * _FPC)])
        plsc.subcore_barrier()
        return 0

    lax.fori_loop(0, _NPAD // _STRIPE, stripe, 0)


def _scatter_max(msg, dst):
    f = pl.kernel(
        _scatter_body,
        out_type=jax.ShapeDtypeStruct((2, _NPAD * _FPC), jnp.float32),
        mesh=_SC_MESH,
        scratch_types=[
            pltpu.VMEM((_FPC * _NPAD,), jnp.float32),
            pltpu.VMEM((_SCC,), jnp.int32),
            pltpu.VMEM((_SCC * MID,), jnp.float32),
            pltpu.VMEM((_FPC, _NPS), jnp.float32),
            pltpu.VMEM((_FPC, _NPS), jnp.float32),
            pltpu.VMEM((_FPC * _NPS,), jnp.float32),
            pltpu.VMEM((_NPS * _FPC,), jnp.float32),
            pltpu.VMEM_SHARED((16, _FPC, _STRIPE), jnp.float32),
        ],
        compiler_params=pltpu.CompilerParams(
            use_tc_tiling_on_sc=False, needs_layout_passes=False),
    )
    return f(msg.reshape(E * MID), dst).reshape(2, _NPAD, _FPC)


def kernel(x, edge_index, edge_attr, W1_node, b1_node, W1_edge, b1_edge,
           W1_net, b1_net, W1_out, b1_out, W2_node, b2_node, W2_edge, b2_edge,
           W2_net, b2_net, W2_out, b2_out):
    src, dst = edge_index[0], edge_index[1]

    h1 = _node_proj(x, W1_node, b1_node)
    g1 = _gather_rows(h1, src)
    msg1 = _edge_mlp(g1, edge_attr, W1_edge, b1_edge, W1_net, b1_net)
    parts1 = _scatter_max(msg1, dst)
    h2 = _post_agg(parts1, W1_out, b1_out, W2_node, b2_node, mode=0)

    g2 = _gather_rows(h2, src)
    msg2 = _edge_mlp(g2, edge_attr, W2_edge, b2_edge, W2_net, b2_net)
    parts2 = _scatter_max(msg2, dst)
    out = _post_agg(parts2, W2_out, b2_out, W2_out, b2_out, mode=1)
    return out


# speedup vs baseline: 4.7005x; 1.0355x over previous
import functools
import math

import jax
import jax.numpy as jnp
from jax import lax
from jax.experimental import pallas as pl
from jax.experimental.pallas import tpu as pltpu
from jax.experimental.pallas import tpu_sc as plsc

N = 10000
E = 320000
D_IN = 128
MID = 16
OUT = 16
NEG = -jnp.inf


def _elu(v):
    return jnp.where(v > 0, v, jnp.exp(jnp.minimum(v, 0.0)) - 1.0)


def _k1_body(x_ref, w_ref, b_ref, o_ref):
    o_ref[...] = (
        jnp.dot(x_ref[...], w_ref[...], preferred_element_type=jnp.float32)
        + b_ref[...]
    )


def _node_proj(x, Wn, bn):
    B = 2000
    return pl.pallas_call(
        _k1_body,
        grid=(N // B,),
        in_specs=[
            pl.BlockSpec((B, D_IN), lambda i: (i, 0)),
            pl.BlockSpec((D_IN, MID), lambda i: (0, 0)),
            pl.BlockSpec((1, MID), lambda i: (0, 0)),
        ],
        out_specs=pl.BlockSpec((B, MID), lambda i: (i, 0)),
        out_shape=jax.ShapeDtypeStruct((N, MID), jnp.float32),
    )(x, Wn, bn.reshape(1, MID))


def _k2_body(g_ref, a_ref, we_ref, be_ref, wm_ref, bm_ref, o_ref):
    t = (
        g_ref[...]
        + jnp.dot(a_ref[...], we_ref[...], preferred_element_type=jnp.float32)
        + be_ref[...]
    )
    t = jnp.maximum(t, 0.0)
    o_ref[...] = (
        jnp.dot(t, wm_ref[...], preferred_element_type=jnp.float32) + bm_ref[...]
    )


def _edge_mlp(g, attr, We, be, Wm, bm):
    g2 = g.reshape(E // 8, 128)
    a2 = attr.reshape(E // 8, 32)
    eye = jnp.eye(8, dtype=jnp.float32)
    WeB = jnp.einsum("pq,ij->piqj", eye, We).reshape(32, 128)
    WmB = jnp.einsum("pq,ij->piqj", eye, Wm).reshape(128, 128)
    beB = jnp.tile(be, 8).reshape(1, 128)
    bmB = jnp.tile(bm, 8).reshape(1, 128)
    B = 4000
    out = pl.pallas_call(
        _k2_body,
        grid=(E // 8 // B,),
        in_specs=[
            pl.BlockSpec((B, 128), lambda i: (i, 0)),
            pl.BlockSpec((B, 32), lambda i: (i, 0)),
            pl.BlockSpec((32, 128), lambda i: (0, 0)),
            pl.BlockSpec((1, 128), lambda i: (0, 0)),
            pl.BlockSpec((128, 128), lambda i: (0, 0)),
            pl.BlockSpec((1, 128), lambda i: (0, 0)),
        ],
        out_specs=pl.BlockSpec((B, 128), lambda i: (i, 0)),
        out_shape=jax.ShapeDtypeStruct((E // 8, 128), jnp.float32),
    )(g2, a2, WeB, beB, WmB, bmB)
    return out.reshape(E, MID)


def _k3_body(parts_ref, wo_ref, bo_ref, w2_ref, b2_ref, o_ref, *, mode):
    p = parts_ref[...]
    agg = jnp.concatenate([p[0], p[1]], axis=1)
    agg = jnp.where(jnp.isfinite(agg), agg, 0.0)
    out = (
        jnp.dot(agg, wo_ref[...], preferred_element_type=jnp.float32) + bo_ref[...]
    )
    if mode == 0:
        h = _elu(_elu(out))
        o_ref[...] = (
            jnp.dot(h, w2_ref[...], preferred_element_type=jnp.float32)
            + b2_ref[...]
        )
    else:
        m = jnp.max(out, axis=1, keepdims=True)
        s = out - m
        lse = jnp.log(jnp.sum(jnp.exp(s), axis=1, keepdims=True))
        o_ref[...] = s - lse


def _post_agg(parts, Wo, bo, W2, b2, mode):
    B = 2000
    dout = W2.shape[1] if mode == 0 else OUT
    return pl.pallas_call(
        functools.partial(_k3_body, mode=mode),
        grid=(N // B,),
        in_specs=[
            pl.BlockSpec((2, B, MID // 2), lambda i: (0, i, 0)),
            pl.BlockSpec((MID, MID), lambda i: (0, 0)),
            pl.BlockSpec((1, MID), lambda i: (0, 0)),
            pl.BlockSpec((MID, dout), lambda i: (0, 0)),
            pl.BlockSpec((1, dout), lambda i: (0, 0)),
        ],
        out_specs=pl.BlockSpec((B, dout), lambda i: (i, 0)),
        out_shape=jax.ShapeDtypeStruct((N, dout), jnp.float32),
    )(parts, Wo, bo.reshape(1, MID), W2, b2.reshape(1, dout))


_SC_MESH = plsc.VectorSubcoreMesh(core_axis_name="c", subcore_axis_name="s")
_NW = 32
_EPW = E // _NW
_GC = 2000


def _gather_body(table_hbm, src_hbm, out_hbm, idx_v, rows_v, sem):
    wid = lax.axis_index("s") * 2 + lax.axis_index("c")
    base = wid * _EPW

    def chunk(i, _):
        off = base + i * _GC
        pltpu.sync_copy(src_hbm.at[pl.ds(off, _GC)], idx_v)
        pltpu.async_copy(table_hbm.at[idx_v], rows_v, sem).wait()
        pltpu.sync_copy(rows_v, out_hbm.at[pl.ds(off, _GC)])
        return 0

    lax.fori_loop(0, _EPW // _GC, chunk, 0)


def _gather_rows(table, src):
    f = pl.kernel(
        _gather_body,
        out_type=jax.ShapeDtypeStruct((E, MID), jnp.float32),
        mesh=_SC_MESH,
        scratch_types=[
            pltpu.VMEM((_GC,), jnp.int32),
            pltpu.VMEM((_GC, MID), jnp.float32),
            pltpu.SemaphoreType.DMA,
        ],
        compiler_params=pltpu.CompilerParams(use_tc_tiling_on_sc=False),
    )
    return f(table, src)


_NPAD = 10240
_STRIPE = 1024
_NPS = _STRIPE // 16
_SCC = 800
_EPT = E // 16
_FPC = MID // 2


def _scatter_body(msg_hbm, dst_hbm, out_hbm, p0, p1, p2, p3, p4, p5, p6, p7,
                  tag_v, dstc_v, msgc_v, accf_v, redf_v, accfl_v, rm_v,
                  shared):
    parts = (p0, p1, p2, p3, p4, p5, p6, p7)
    c = lax.axis_index("c")
    sid = lax.axis_index("s")
    base = sid * _EPT
    iot = lax.iota(jnp.int32, 16)
    vneg = jnp.full((16,), NEG, jnp.float32)

    def initcol(k, _):
        for f in range(_FPC):
            parts[f][pl.ds(k * 16, 16)] = vneg
        return 0
    lax.fori_loop(0, _NPAD // 16, initcol, 0)

    def chunk(i, _):
        off = base + i * _SCC
        pltpu.sync_copy(dst_hbm.at[pl.ds(off, _SCC)], dstc_v)
        pltpu.sync_copy(msg_hbm.at[pl.ds(off * MID, _SCC * MID)], msgc_v)

        def group(g, _):
            dvec = dstc_v[pl.ds(g * 16, 16)]
            uid = iot + g * 16
            eb = uid * MID + c * _FPC
            msgf = [plsc.load_gather(msgc_v, [eb + f]) for f in range(_FPC)]
            plsc.store_scatter(tag_v, [dvec], uid)
            rb = plsc.load_gather(tag_v, [dvec])
            cnt = plsc.all_reduce_population_count(rb != uid)

            def fast():
                for f in range(_FPC):
                    cur = plsc.load_gather(parts[f], [dvec])
                    plsc.store_scatter(parts[f], [dvec],
                                       jnp.maximum(cur, msgf[f]))

            def slow():
                def lane(l, _):
                    m = iot == l
                    for f in range(_FPC):
                        cur = plsc.load_gather(parts[f], [dvec])
                        plsc.store_scatter(parts[f], [dvec],
                                           jnp.maximum(cur, msgf[f]), mask=m)
                    return 0
                lax.fori_loop(0, 16, lane, 0)

            lax.cond(cnt[0] > 0, slow, fast)
            return 0

        lax.fori_loop(0, _SCC // 16, group, 0)
        return 0

    lax.fori_loop(0, _EPT // _SCC, chunk, 0)

    idx_t = (iot % _FPC) * _NPS + iot // _FPC

    def stripe(r, _):
        for f in range(_FPC):
            pltpu.sync_copy(
                parts[f].at[pl.ds(r * _STRIPE, _STRIPE)],
                shared.at[sid, f])
        plsc.subcore_barrier()
        n0 = sid * _NPS
        pltpu.sync_copy(shared.at[0, :, pl.ds(n0, _NPS)], accf_v)

        def tred(t, _):
            pltpu.sync_copy(shared.at[t, :, pl.ds(n0, _NPS)], redf_v)
            for f in range(_FPC):
                for q in range(_NPS // 16):
                    sl = pl.ds(q * 16, 16)
                    accf_v[f, sl] = jnp.maximum(accf_v[f, sl], redf_v[f, sl])
            return 0
        lax.fori_loop(1, 16, tred, 0)

        for f in range(_FPC):
            for q in range(_NPS // 16):
                accfl_v[pl.ds(f * _NPS + q * 16, 16)] = \
                    accf_v[f, pl.ds(q * 16, 16)]
        for j in range(_NPS // 2):
            rm_v[pl.ds(j * 16, 16)] = plsc.load_gather(
                accfl_v, [idx_t + 2 * j])
        pltpu.sync_copy(
            rm_v,
            out_hbm.at[c, pl.ds((r * _STRIPE + n0) * _FPC, _NPS * _FPC)])
        plsc.subcore_barrier()
        return 0

    lax.fori_loop(0, _NPAD // _STRIPE, stripe, 0)


def _scatter_max(msg, dst):
    f = pl.kernel(
        _scatter_body,
        out_type=jax.ShapeDtypeStruct((2, _NPAD * _FPC), jnp.float32),
        mesh=_SC_MESH,
        scratch_types=[
            pltpu.VMEM((_NPAD,), jnp.float32),
            pltpu.VMEM((_NPAD,), jnp.float32),
            pltpu.VMEM((_NPAD,), jnp.float32),
            pltpu.VMEM((_NPAD,), jnp.float32),
            pltpu.VMEM((_NPAD,), jnp.float32),
            pltpu.VMEM((_NPAD,), jnp.float32),
            pltpu.VMEM((_NPAD,), jnp.float32),
            pltpu.VMEM((_NPAD,), jnp.float32),
            pltpu.VMEM((_NPAD,), jnp.int32),
            pltpu.VMEM((_SCC,), jnp.int32),
            pltpu.VMEM((_SCC * MID,), jnp.float32),
            pltpu.VMEM((_FPC, _NPS), jnp.float32),
            pltpu.VMEM((_FPC, _NPS), jnp.float32),
            pltpu.VMEM((_FPC * _NPS,), jnp.float32),
            pltpu.VMEM((_NPS * _FPC,), jnp.float32),
            pltpu.VMEM_SHARED((16, _FPC, _STRIPE), jnp.float32),
        ],
        compiler_params=pltpu.CompilerParams(
            use_tc_tiling_on_sc=False, needs_layout_passes=False),
    )
    return f(msg.reshape(E * MID), dst).reshape(2, _NPAD, _FPC)


def kernel(x, edge_index, edge_attr, W1_node, b1_node, W1_edge, b1_edge,
           W1_net, b1_net, W1_out, b1_out, W2_node, b2_node, W2_edge, b2_edge,
           W2_net, b2_net, W2_out, b2_out):
    src, dst = edge_index[0], edge_index[1]

    h1 = _node_proj(x, W1_node, b1_node)
    g1 = _gather_rows(h1, src)
    msg1 = _edge_mlp(g1, edge_attr, W1_edge, b1_edge, W1_net, b1_net)
    parts1 = _scatter_max(msg1, dst)
    h2 = _post_agg(parts1, W1_out, b1_out, W2_node, b2_node, mode=0)

    g2 = _gather_rows(h2, src)
    msg2 = _edge_mlp(g2, edge_attr, W2_edge, b2_edge, W2_net, b2_net)
    parts2 = _scatter_max(msg2, dst)
    out = _post_agg(parts2, W2_out, b2_out, W2_out, b2_out, mode=1)
    return out
